# baseline (speedup 1.0000x reference)
import functools

import jax
import jax.numpy as jnp
from jax import lax
from jax.experimental import pallas as pl
from jax.experimental.pallas import tpu as pltpu
from jax.experimental.pallas import tpu_sc as plsc


def _make_sc_gather(N):
    info = plsc.get_sparse_core_info()
    NC, NS = info.num_cores, info.num_subcores
    n_per_w = N // (NC * NS)

    mesh = plsc.VectorSubcoreMesh(core_axis_name="c", subcore_axis_name="s")

    @functools.partial(
        pl.kernel,
        mesh=mesh,
        out_type=jax.ShapeDtypeStruct((N,), jnp.float32),
        scratch_types=[
            pltpu.VMEM((n_per_w,), jnp.int32),
            pltpu.VMEM((n_per_w,), jnp.float32),
            pltpu.SemaphoreType.DMA,
        ],
        compiler_params=pltpu.CompilerParams(use_tc_tiling_on_sc=False),
    )
    def gather_kernel(table_hbm, idx_hbm, out_hbm, idx_v, vals_v, sem):
        wid = lax.axis_index("s") * NC + lax.axis_index("c")
        base = wid * n_per_w
        pltpu.sync_copy(idx_hbm.at[pl.ds(base, n_per_w)], idx_v)
        pltpu.async_copy(table_hbm.at[idx_v], vals_v, sem).wait()
        pltpu.sync_copy(vals_v, out_hbm.at[pl.ds(base, n_per_w)])

    return gather_kernel


def _mm_body(a_ref, wt_ref, b_ref, o_ref):
    a = jnp.maximum(a_ref[...], 0.0)
    h = a.shape[0]
    ones = jnp.ones((h, 1), jnp.float32)
    wtb = jnp.concatenate([wt_ref[...], b_ref[...]], axis=0)
    for k in (0, 1):
        a1 = jnp.concatenate([a[:, 64 * k : 64 * (k + 1)], ones], axis=1)
        o_ref[:, h * k : h * (k + 1)] = lax.dot_general(
            wtb,
            a1,
            dimension_numbers=(((0,), (1,)), ((), ())),
            preferred_element_type=jnp.float32,
        )


def kernel(x, emb_table, W, b):
    B, C = x.shape
    V, E = emb_table.shape
    F = C * E

    idx = (
        x.astype(jnp.int32)
        .reshape(2, B // 2, C)
        .transpose(1, 0, 2)
        .reshape(-1)
    )
    idx2 = (idx[:, None] * E + jnp.arange(E, dtype=jnp.int32)).reshape(-1)
    gather = _make_sc_gather(idx2.shape[0])
    acts = gather(emb_table.reshape(-1), idx2).reshape(B // 2, 2 * F)

    VT = 2048
    nsteps = pl.cdiv(V, VT)
    outT = pl.pallas_call(
        _mm_body,
        grid=(nsteps,),
        in_specs=[
            pl.BlockSpec((B // 2, 2 * F), lambda i: (0, 0)),
            pl.BlockSpec((F, VT), lambda i: (0, i)),
            pl.BlockSpec((1, VT), lambda i: (0, i)),
        ],
        out_specs=pl.BlockSpec((VT, B), lambda i: (i, 0)),
        out_shape=jax.ShapeDtypeStruct((V, B), jnp.float32),
    )(acts, W.T, b.reshape(1, V))
    return outT.T


# speedup vs baseline: 1.0168x; 1.0168x over previous
import functools

import jax
import jax.numpy as jnp
from jax import lax
from jax.experimental import pallas as pl
from jax.experimental.pallas import tpu as pltpu
from jax.experimental.pallas import tpu_sc as plsc


def _make_sc_gather(V, D, B):
    info = plsc.get_sparse_core_info()
    NC, NS = info.num_cores, info.num_subcores
    NW = NC * NS
    b_per_w = B // NW

    mesh = plsc.VectorSubcoreMesh(core_axis_name="c", subcore_axis_name="s")

    @functools.partial(
        pl.kernel,
        mesh=mesh,
        out_type=jax.ShapeDtypeStruct((B, D), jnp.float32),
        scratch_types=[
            pltpu.VMEM((b_per_w,), jnp.int32),
            pltpu.VMEM((b_per_w, D), jnp.float32),
            pltpu.SemaphoreType.DMA,
        ],
        compiler_params=pltpu.CompilerParams(use_tc_tiling_on_sc=False),
    )
    def gather_kernel(table_hbm, idx_hbm, out_hbm, idx_v, rows_v, sem):
        wid = lax.axis_index("s") * NC + lax.axis_index("c")
        base = wid * b_per_w
        pltpu.sync_copy(idx_hbm.at[pl.ds(base, b_per_w)], idx_v)
        pltpu.async_copy(table_hbm.at[idx_v], rows_v, sem).wait()
        pltpu.sync_copy(rows_v, out_hbm.at[pl.ds(base, b_per_w)])

    return gather_kernel


def _mm_body(a_ref, wt_ref, b_ref, o_ref):
    a = jnp.maximum(a_ref[...], 0.0)
    a1 = jnp.concatenate([a, jnp.ones((a.shape[0], 1), jnp.float32)], axis=1)
    wtb = jnp.concatenate([wt_ref[...], b_ref[...]], axis=0)
    o_ref[...] = lax.dot_general(
        wtb,
        a1,
        dimension_numbers=(((0,), (1,)), ((), ())),
        preferred_element_type=jnp.float32,
    )


def kernel(x, emb_table, W, b):
    B, C = x.shape
    V, E = emb_table.shape
    F = C * E

    idx = x.reshape(-1).astype(jnp.int32)
    gather = _make_sc_gather(V, E, idx.shape[0])
    acts = gather(emb_table, idx).reshape(B, F)

    VT = 2048
    nsteps = pl.cdiv(V, VT)
    outT = pl.pallas_call(
        _mm_body,
        grid=(nsteps,),
        in_specs=[
            pl.BlockSpec((B, F), lambda i: (0, 0)),
            pl.BlockSpec((F, VT), lambda i: (0, i)),
            pl.BlockSpec((1, VT), lambda i: (0, i)),
        ],
        out_specs=pl.BlockSpec((VT, B), lambda i: (i, 0)),
        out_shape=jax.ShapeDtypeStruct((V, B), jnp.float32),
    )(acts, W.T, b.reshape(1, V))
    return outT.T
